# baseline (speedup 1.0000x reference)
import functools

import jax
import jax.numpy as jnp
from jax import lax
from jax.experimental import pallas as pl
from jax.experimental.pallas import tpu as pltpu
from jax.experimental.pallas import tpu_sc as plsc

B = 16384
DIM = 16
NC = 2
NS = 16
NW = NC * NS
BPW = B // NW
CHUNK = 128
NCHUNK = BPW // CHUNK


def _body(u_hbm, i_hbm, n_hbm, user_hbm, item_hbm, pos_hbm, neg_hbm,
          idx_u, idx_i, idx_n, u_rows, i_rows, n_rows, pos_v, neg_v, sem):
    wid = lax.axis_index("s") * NC + lax.axis_index("c")

    pltpu.sync_copy(u_hbm.at[pl.ds(wid * NCHUNK, NCHUNK)], idx_u)
    pltpu.sync_copy(i_hbm.at[pl.ds(wid * NCHUNK, NCHUNK)], idx_i)
    pltpu.sync_copy(n_hbm.at[pl.ds(wid * NCHUNK, NCHUNK)], idx_n)

    copies = []
    for j in range(NCHUNK):
        sl = pl.ds(j * CHUNK, CHUNK)
        copies.append(pltpu.async_copy(user_hbm.at[idx_u.at[j]], u_rows.at[sl], sem))
        copies.append(pltpu.async_copy(item_hbm.at[idx_i.at[j]], i_rows.at[sl], sem))
        copies.append(pltpu.async_copy(item_hbm.at[idx_n.at[j]], n_rows.at[sl], sem))
    for c in copies:
        c.wait()

    def group(g, carry):
        rows = g * DIM + lax.iota(jnp.int32, 16)
        accp = jnp.zeros((16,), jnp.float32)
        accn = jnp.zeros((16,), jnp.float32)
        for d in range(DIM):
            col = jnp.full((16,), d, jnp.int32)
            uc = plsc.load_gather(u_rows, [rows, col])
            ic = plsc.load_gather(i_rows, [rows, col])
            nc = plsc.load_gather(n_rows, [rows, col])
            accp = accp + uc * ic
            accn = accn + uc * nc
        pos_v[pl.ds(g * DIM, 16)] = accp
        neg_v[pl.ds(g * DIM, 16)] = accn
        return carry

    lax.fori_loop(0, BPW // DIM, group, 0)

    pltpu.sync_copy(pos_v, pos_hbm.at[pl.ds(wid * BPW, BPW)])
    pltpu.sync_copy(neg_v, neg_hbm.at[pl.ds(wid * BPW, BPW)])


@jax.jit
def kernel(u, i, neg_i, user_emb, item_emb):
    u2 = u.astype(jnp.int32).reshape(NW * NCHUNK, CHUNK)
    i2 = i.astype(jnp.int32).reshape(NW * NCHUNK, CHUNK)
    n2 = neg_i.astype(jnp.int32).reshape(NW * NCHUNK, CHUNK)

    mesh = plsc.VectorSubcoreMesh(core_axis_name="c", subcore_axis_name="s",
                                  num_cores=NC, num_subcores=NS)
    run = pl.kernel(
        _body,
        out_type=(jax.ShapeDtypeStruct((B,), jnp.float32),
                  jax.ShapeDtypeStruct((B,), jnp.float32)),
        mesh=mesh,
        scratch_types=[
            pltpu.VMEM((NCHUNK, CHUNK), jnp.int32),
            pltpu.VMEM((NCHUNK, CHUNK), jnp.int32),
            pltpu.VMEM((NCHUNK, CHUNK), jnp.int32),
            pltpu.VMEM((BPW, DIM), jnp.float32),
            pltpu.VMEM((BPW, DIM), jnp.float32),
            pltpu.VMEM((BPW, DIM), jnp.float32),
            pltpu.VMEM((BPW,), jnp.float32),
            pltpu.VMEM((BPW,), jnp.float32),
            pltpu.SemaphoreType.DMA,
        ],
        compiler_params=pltpu.CompilerParams(needs_layout_passes=False,
                                             use_tc_tiling_on_sc=False),
    )
    return run(u2, i2, n2, user_emb, item_emb)
